# baseline (speedup 1.0000x reference)
import functools

import jax
import jax.numpy as jnp
from jax import lax
from jax.experimental import pallas as pl
from jax.experimental.pallas import tpu as pltpu
from jax.experimental.pallas import tpu_sc as plsc

BATCH = 16384
FIELDS = 100
EMBED_DIM = 32
NROWS = BATCH * FIELDS
OUT_MINOR = 128
OUT_MAJOR = NROWS * EMBED_DIM // OUT_MINOR

NUM_CORES = 2
NUM_SUBCORES = 16
NUM_WORKERS = NUM_CORES * NUM_SUBCORES
ROWS_PER_WORKER = NROWS // NUM_WORKERS
CHUNK = 1600
NUM_CHUNKS = ROWS_PER_WORKER // CHUNK
NUM_PAIRS = NUM_CHUNKS // 2
CHUNK_OUT = CHUNK * EMBED_DIM // OUT_MINOR
LANES = 16
CLS = OUT_MINOR // EMBED_DIM
PER_CLS = CHUNK // CLS


def _make_gather():
    mesh = plsc.VectorSubcoreMesh(core_axis_name="c", subcore_axis_name="s")

    @functools.partial(
        pl.kernel,
        mesh=mesh,
        out_type=jax.ShapeDtypeStruct((OUT_MAJOR, OUT_MINOR), jnp.float32),
        scratch_types=[
            pltpu.VMEM((CHUNK,), jnp.int32),
            pltpu.VMEM((CHUNK,), jnp.int32),
            pltpu.VMEM((2, CLS, PER_CLS), jnp.int32),
            pltpu.VMEM((2, CLS, PER_CLS, EMBED_DIM), jnp.float32),
            pltpu.SemaphoreType.DMA,
            pltpu.SemaphoreType.DMA,
            pltpu.SemaphoreType.DMA,
            pltpu.SemaphoreType.DMA,
            pltpu.SemaphoreType.DMA,
            pltpu.SemaphoreType.DMA,
        ],
        compiler_params=pltpu.CompilerParams(use_tc_tiling_on_sc=False),
    )
    def gather_kernel(idx_hbm, table_hbm, out_hbm, idx_v0, idx_v1, cls_v, rows_v,
                      idx_sem0, idx_sem1, gat_sem0, gat_sem1,
                      st_sem0, st_sem1):
        wid = lax.axis_index("s") * NUM_CORES + lax.axis_index("c")
        base = wid * ROWS_PER_WORKER
        obase = wid * (ROWS_PER_WORKER * EMBED_DIM // OUT_MINOR)
        idx_vs = (idx_v0, idx_v1)
        idx_sems = (idx_sem0, idx_sem1)
        gat_sems = (gat_sem0, gat_sem1)
        st_sems = (st_sem0, st_sem1)
        lane = lax.iota(jnp.int32, LANES)
        perms = [(lane & (CLS - 1)) * CLS + k for k in range(CLS)]
        blocks = [(lane >> 2) == m for m in range(CLS)]

        for s in (0, 1):
            pltpu.async_copy(
                idx_hbm.at[pl.ds(base + s * CHUNK, CHUNK)], idx_vs[s],
                idx_sems[s])

        def body(gpair, carry):
            for s in (0, 1):
                g = gpair * 2 + s
                f0 = base + g * CHUNK
                o0 = obase + g * CHUNK_OUT

                pltpu.make_async_copy(
                    idx_hbm.at[pl.ds(f0, CHUNK)], idx_vs[s],
                    idx_sems[s]).wait()

                for j in range(CHUNK // (LANES * CLS)):
                    vs_ = [idx_vs[s][pl.ds((j * CLS + m) * LANES, LANES)]
                           for m in range(CLS)]
                    for k in range(CLS):
                        gk = [vs_[m][perms[k]] for m in range(CLS)]
                        merged = gk[0]
                        for m in range(1, CLS):
                            merged = jnp.where(blocks[m], gk[m], merged)
                        cls_v[s, k, pl.ds(j * LANES, LANES)] = merged

                @pl.when(gpair >= 1)
                def _():
                    for k in range(CLS):
                        pltpu.make_async_copy(
                            rows_v.at[s, k],
                            out_hbm.at[pl.ds(o0 - 2 * CHUNK_OUT, CHUNK_OUT),
                                       pl.ds(EMBED_DIM * k, EMBED_DIM)],
                            st_sems[s]).wait()

                copies = [
                    pltpu.async_copy(
                        table_hbm.at[cls_v.at[s, k]],
                        rows_v.at[s, k],
                        gat_sems[s])
                    for k in range(CLS)
                ]
                for c in copies:
                    c.wait()

                for k in range(CLS):
                    pltpu.async_copy(
                        rows_v.at[s, k],
                        out_hbm.at[pl.ds(o0, CHUNK_OUT),
                                   pl.ds(EMBED_DIM * k, EMBED_DIM)],
                        st_sems[s])

                @pl.when(gpair < NUM_PAIRS - 1)
                def _():
                    pltpu.async_copy(
                        idx_hbm.at[pl.ds(f0 + 2 * CHUNK, CHUNK)],
                        idx_vs[s], idx_sems[s])

            return carry

        lax.fori_loop(0, NUM_PAIRS, body, 0)

        for s in (0, 1):
            g = NUM_CHUNKS - 2 + s
            for k in range(CLS):
                pltpu.make_async_copy(
                    rows_v.at[s, k],
                    out_hbm.at[pl.ds(obase + g * CHUNK_OUT, CHUNK_OUT),
                               pl.ds(EMBED_DIM * k, EMBED_DIM)],
                    st_sems[s]).wait()

    return gather_kernel


_gather = _make_gather()

TC_BB = 512
TC_NBLK = BATCH // TC_BB
TC_F4 = FIELDS // 4


def _tc_transpose_body(x_ref, o_ref):
    o_ref[...] = jnp.transpose(
        x_ref[...].reshape(TC_BB, 4, EMBED_DIM), (1, 2, 0))


_tc_transpose = pl.pallas_call(
    _tc_transpose_body,
    grid=(TC_F4, TC_NBLK),
    in_specs=[pl.BlockSpec((TC_BB, 128), lambda f, i: (i, f))],
    out_specs=pl.BlockSpec((4, EMBED_DIM, TC_BB), lambda f, i: (f, 0, i)),
    out_shape=jax.ShapeDtypeStruct((FIELDS, EMBED_DIM, BATCH), jnp.float32),
)


def kernel(indices, table):
    out = _gather(indices.reshape(NROWS), table)
    mid = _tc_transpose(out.reshape(BATCH, FIELDS * EMBED_DIM))
    return mid.transpose(2, 0, 1)


# speedup vs baseline: 3.3204x; 3.3204x over previous
import functools

import jax
import jax.numpy as jnp
from jax import lax
from jax.experimental import pallas as pl
from jax.experimental.pallas import tpu as pltpu
from jax.experimental.pallas import tpu_sc as plsc

BATCH = 16384
FIELDS = 100
EMBED_DIM = 32
NROWS = BATCH * FIELDS
OUT_MINOR = 128
OUT_MAJOR = NROWS * EMBED_DIM // OUT_MINOR

NUM_CORES = 2
NUM_SUBCORES = 16
NUM_WORKERS = NUM_CORES * NUM_SUBCORES
ROWS_PER_WORKER = NROWS // NUM_WORKERS
CHUNK = 1600
NUM_CHUNKS = ROWS_PER_WORKER // CHUNK
NUM_PAIRS = NUM_CHUNKS // 2
CHUNK_OUT = CHUNK * EMBED_DIM // OUT_MINOR
LANES = 16
CLS = OUT_MINOR // EMBED_DIM
PER_CLS = CHUNK // CLS


def _make_gather():
    mesh = plsc.VectorSubcoreMesh(core_axis_name="c", subcore_axis_name="s")

    @functools.partial(
        pl.kernel,
        mesh=mesh,
        out_type=jax.ShapeDtypeStruct((OUT_MAJOR, OUT_MINOR), jnp.float32),
        scratch_types=[
            pltpu.VMEM((CHUNK,), jnp.int32),
            pltpu.VMEM((CHUNK,), jnp.int32),
            pltpu.VMEM((2, CLS, PER_CLS), jnp.int32),
            pltpu.VMEM((2, CLS, PER_CLS, EMBED_DIM), jnp.float32),
            pltpu.SemaphoreType.DMA,
            pltpu.SemaphoreType.DMA,
            pltpu.SemaphoreType.DMA,
            pltpu.SemaphoreType.DMA,
            pltpu.SemaphoreType.DMA,
            pltpu.SemaphoreType.DMA,
        ],
        compiler_params=pltpu.CompilerParams(use_tc_tiling_on_sc=False),
    )
    def gather_kernel(idx_hbm, table_hbm, out_hbm, idx_v0, idx_v1, cls_v, rows_v,
                      idx_sem0, idx_sem1, gat_sem0, gat_sem1,
                      st_sem0, st_sem1):
        wid = lax.axis_index("s") * NUM_CORES + lax.axis_index("c")
        base = wid * ROWS_PER_WORKER
        obase = wid * (ROWS_PER_WORKER * EMBED_DIM // OUT_MINOR)
        idx_vs = (idx_v0, idx_v1)
        idx_sems = (idx_sem0, idx_sem1)
        gat_sems = (gat_sem0, gat_sem1)
        st_sems = (st_sem0, st_sem1)
        lane = lax.iota(jnp.int32, LANES)
        perms = [(lane & (CLS - 1)) * CLS + k for k in range(CLS)]
        blocks = [(lane >> 2) == m for m in range(CLS)]

        for s in (0, 1):
            pltpu.async_copy(
                idx_hbm.at[pl.ds(base + s * CHUNK, CHUNK)], idx_vs[s],
                idx_sems[s])

        def body(gpair, carry):
            for s in (0, 1):
                g = gpair * 2 + s
                f0 = base + g * CHUNK
                o0 = obase + g * CHUNK_OUT

                pltpu.make_async_copy(
                    idx_hbm.at[pl.ds(f0, CHUNK)], idx_vs[s],
                    idx_sems[s]).wait()

                for j in range(CHUNK // (LANES * CLS)):
                    vs_ = [idx_vs[s][pl.ds((j * CLS + m) * LANES, LANES)]
                           for m in range(CLS)]
                    for k in range(CLS):
                        gk = [vs_[m][perms[k]] for m in range(CLS)]
                        merged = gk[0]
                        for m in range(1, CLS):
                            merged = jnp.where(blocks[m], gk[m], merged)
                        cls_v[s, k, pl.ds(j * LANES, LANES)] = merged

                @pl.when(gpair >= 1)
                def _():
                    for k in range(CLS):
                        pltpu.make_async_copy(
                            rows_v.at[s, k],
                            out_hbm.at[pl.ds(o0 - 2 * CHUNK_OUT, CHUNK_OUT),
                                       pl.ds(EMBED_DIM * k, EMBED_DIM)],
                            st_sems[s]).wait()

                copies = [
                    pltpu.async_copy(
                        table_hbm.at[cls_v.at[s, k]],
                        rows_v.at[s, k],
                        gat_sems[s])
                    for k in range(CLS)
                ]
                for c in copies:
                    c.wait()

                for k in range(CLS):
                    pltpu.async_copy(
                        rows_v.at[s, k],
                        out_hbm.at[pl.ds(o0, CHUNK_OUT),
                                   pl.ds(EMBED_DIM * k, EMBED_DIM)],
                        st_sems[s])

                @pl.when(gpair < NUM_PAIRS - 1)
                def _():
                    pltpu.async_copy(
                        idx_hbm.at[pl.ds(f0 + 2 * CHUNK, CHUNK)],
                        idx_vs[s], idx_sems[s])

            return carry

        lax.fori_loop(0, NUM_PAIRS, body, 0)

        for s in (0, 1):
            g = NUM_CHUNKS - 2 + s
            for k in range(CLS):
                pltpu.make_async_copy(
                    rows_v.at[s, k],
                    out_hbm.at[pl.ds(obase + g * CHUNK_OUT, CHUNK_OUT),
                               pl.ds(EMBED_DIM * k, EMBED_DIM)],
                    st_sems[s]).wait()

    return gather_kernel


_gather = _make_gather()

TC_BB = 512
TC_NBLK = BATCH // TC_BB
TC_F4 = FIELDS // 4


def _tc_transpose_body(x_ref, o_ref):
    o_ref[...] = x_ref[...].T.reshape(4, EMBED_DIM, TC_BB)


_tc_transpose = pl.pallas_call(
    _tc_transpose_body,
    grid=(TC_F4, TC_NBLK),
    in_specs=[pl.BlockSpec((TC_BB, 128), lambda f, i: (i, f))],
    out_specs=pl.BlockSpec((4, EMBED_DIM, TC_BB), lambda f, i: (f, 0, i)),
    out_shape=jax.ShapeDtypeStruct((FIELDS, EMBED_DIM, BATCH), jnp.float32),
)


def kernel(indices, table):
    out = _gather(indices.reshape(NROWS), table)
    mid = _tc_transpose(out.reshape(BATCH, FIELDS * EMBED_DIM))
    return mid.transpose(2, 0, 1)


# speedup vs baseline: 3.9080x; 1.1770x over previous
import functools

import jax
import jax.numpy as jnp
from jax import lax
from jax.experimental import pallas as pl
from jax.experimental.pallas import tpu as pltpu
from jax.experimental.pallas import tpu_sc as plsc

BATCH = 16384
FIELDS = 100
EMBED_DIM = 32
NROWS = BATCH * FIELDS
NUM_COLUMNS_P1 = 1000001
OUT_MINOR = 128
OUT_MAJOR = NROWS * EMBED_DIM // OUT_MINOR

NUM_CORES = 2
NUM_SUBCORES = 16
NUM_WORKERS = NUM_CORES * NUM_SUBCORES
ROWS_PER_WORKER = NROWS // NUM_WORKERS
CHUNK = 1600
NUM_CHUNKS = ROWS_PER_WORKER // CHUNK
NUM_PAIRS = NUM_CHUNKS // 2
CHUNK_OUT = CHUNK * EMBED_DIM // OUT_MINOR
LANES = 16
CLS = OUT_MINOR // EMBED_DIM
PER_CLS = CHUNK // CLS


def _make_gather():
    mesh = plsc.VectorSubcoreMesh(core_axis_name="c", subcore_axis_name="s")

    @functools.partial(
        pl.kernel,
        mesh=mesh,
        out_type=jax.ShapeDtypeStruct((OUT_MAJOR, OUT_MINOR), jnp.float32),
        scratch_types=[
            pltpu.VMEM((CHUNK,), jnp.int32),
            pltpu.VMEM((CHUNK,), jnp.int32),
            pltpu.VMEM((2, CLS, PER_CLS), jnp.int32),
            pltpu.VMEM((2, CLS, PER_CLS, EMBED_DIM), jnp.float32),
            pltpu.SemaphoreType.DMA,
            pltpu.SemaphoreType.DMA,
            pltpu.SemaphoreType.DMA,
            pltpu.SemaphoreType.DMA,
            pltpu.SemaphoreType.DMA,
            pltpu.SemaphoreType.DMA,
        ],
        compiler_params=pltpu.CompilerParams(use_tc_tiling_on_sc=False),
    )
    def gather_kernel(idx_hbm, table_hbm, out_hbm, idx_v0, idx_v1, cls_v, rows_v,
                      idx_sem0, idx_sem1, gat_sem0, gat_sem1,
                      st_sem0, st_sem1):
        wid = lax.axis_index("s") * NUM_CORES + lax.axis_index("c")
        base = wid * ROWS_PER_WORKER
        obase = wid * (ROWS_PER_WORKER * EMBED_DIM // OUT_MINOR)
        idx_vs = (idx_v0, idx_v1)
        idx_sems = (idx_sem0, idx_sem1)
        gat_sems = (gat_sem0, gat_sem1)
        st_sems = (st_sem0, st_sem1)
        lane = lax.iota(jnp.int32, LANES)
        perms = [(lane & (CLS - 1)) * CLS + k for k in range(CLS)]
        blocks = [(lane >> 2) == m for m in range(CLS)]

        for s in (0, 1):
            pltpu.async_copy(
                idx_hbm.at[pl.ds(base + s * CHUNK, CHUNK)], idx_vs[s],
                idx_sems[s])

        def body(gpair, carry):
            for s in (0, 1):
                g = gpair * 2 + s
                f0 = base + g * CHUNK
                o0 = obase + g * CHUNK_OUT

                pltpu.make_async_copy(
                    idx_hbm.at[pl.ds(f0, CHUNK)], idx_vs[s],
                    idx_sems[s]).wait()

                for j in range(CHUNK // (LANES * CLS)):
                    vs_ = [idx_vs[s][pl.ds((j * CLS + m) * LANES, LANES)]
                           for m in range(CLS)]
                    for k in range(CLS):
                        gk = [vs_[m][perms[k]] for m in range(CLS)]
                        merged = gk[0]
                        for m in range(1, CLS):
                            merged = jnp.where(blocks[m], gk[m], merged)
                        cls_v[s, k, pl.ds(j * LANES, LANES)] = merged

                @pl.when(gpair >= 1)
                def _():
                    for k in range(CLS):
                        pltpu.make_async_copy(
                            rows_v.at[s, k],
                            out_hbm.at[pl.ds(o0 - 2 * CHUNK_OUT, CHUNK_OUT),
                                       pl.ds(EMBED_DIM * k, EMBED_DIM)],
                            st_sems[s]).wait()

                copies = [
                    pltpu.async_copy(
                        table_hbm.at[cls_v.at[s, k]],
                        rows_v.at[s, k],
                        gat_sems[s])
                    for k in range(CLS)
                ]
                for c in copies:
                    c.wait()

                for k in range(CLS):
                    pltpu.async_copy(
                        rows_v.at[s, k],
                        out_hbm.at[pl.ds(o0, CHUNK_OUT),
                                   pl.ds(EMBED_DIM * k, EMBED_DIM)],
                        st_sems[s])

                @pl.when(gpair < NUM_PAIRS - 1)
                def _():
                    pltpu.async_copy(
                        idx_hbm.at[pl.ds(f0 + 2 * CHUNK, CHUNK)],
                        idx_vs[s], idx_sems[s])

            return carry

        lax.fori_loop(0, NUM_PAIRS, body, 0)

        for s in (0, 1):
            g = NUM_CHUNKS - 2 + s
            for k in range(CLS):
                pltpu.make_async_copy(
                    rows_v.at[s, k],
                    out_hbm.at[pl.ds(obase + g * CHUNK_OUT, CHUNK_OUT),
                               pl.ds(EMBED_DIM * k, EMBED_DIM)],
                    st_sems[s]).wait()

    return gather_kernel


_gather = _make_gather()

TC_BB = 1024
TC_NBLK = BATCH // TC_BB
TC_F4 = FIELDS // 4


def _tc_transpose_body(x_ref, o_ref):
    o_ref[...] = x_ref[...].T.reshape(4, EMBED_DIM, TC_BB)


_tc_transpose = pl.pallas_call(
    _tc_transpose_body,
    grid=(TC_F4, TC_NBLK),
    in_specs=[pl.BlockSpec((TC_BB, 128), lambda f, i: (i, f))],
    out_specs=pl.BlockSpec((4, EMBED_DIM, TC_BB), lambda f, i: (f, 0, i)),
    out_shape=jax.ShapeDtypeStruct((FIELDS, EMBED_DIM, BATCH), jnp.float32),
)


def kernel(indices, table):
    out = _gather(indices.reshape(NROWS), table)
    mid = _tc_transpose(out.reshape(BATCH, FIELDS * EMBED_DIM))
    return mid.transpose(2, 0, 1)


# speedup vs baseline: 4.2438x; 1.0859x over previous
import functools

import jax
import jax.numpy as jnp
from jax import lax
from jax.experimental import pallas as pl
from jax.experimental.pallas import tpu as pltpu
from jax.experimental.pallas import tpu_sc as plsc

BATCH = 16384
FIELDS = 100
EMBED_DIM = 32
NROWS = BATCH * FIELDS
NUM_COLUMNS_P1 = 1000001
OUT_MINOR = 128
OUT_MAJOR = NROWS * EMBED_DIM // OUT_MINOR

NUM_CORES = 2
NUM_SUBCORES = 16
NUM_WORKERS = NUM_CORES * NUM_SUBCORES
ROWS_PER_WORKER = NROWS // NUM_WORKERS
CHUNK = 1600
NUM_CHUNKS = ROWS_PER_WORKER // CHUNK
NUM_PAIRS = NUM_CHUNKS // 2
CHUNK_OUT = CHUNK * EMBED_DIM // OUT_MINOR
LANES = 16
CLS = OUT_MINOR // EMBED_DIM
PER_CLS = CHUNK // CLS


def _make_gather():
    mesh = plsc.VectorSubcoreMesh(core_axis_name="c", subcore_axis_name="s")

    @functools.partial(
        pl.kernel,
        mesh=mesh,
        out_type=jax.ShapeDtypeStruct((OUT_MAJOR, OUT_MINOR), jnp.float32),
        scratch_types=[
            pltpu.VMEM((CHUNK,), jnp.int32),
            pltpu.VMEM((CHUNK,), jnp.int32),
            pltpu.VMEM((2, CLS, PER_CLS), jnp.int32),
            pltpu.VMEM((2, CLS, PER_CLS, EMBED_DIM), jnp.float32),
            pltpu.SemaphoreType.DMA,
            pltpu.SemaphoreType.DMA,
            pltpu.SemaphoreType.DMA,
            pltpu.SemaphoreType.DMA,
            pltpu.SemaphoreType.DMA,
            pltpu.SemaphoreType.DMA,
        ],
        compiler_params=pltpu.CompilerParams(use_tc_tiling_on_sc=False),
    )
    def gather_kernel(idx_hbm, table_hbm, out_hbm, idx_v0, idx_v1, cls_v, rows_v,
                      idx_sem0, idx_sem1, gat_sem0, gat_sem1,
                      st_sem0, st_sem1):
        wid = lax.axis_index("s") * NUM_CORES + lax.axis_index("c")
        base = wid * ROWS_PER_WORKER
        obase = wid * (ROWS_PER_WORKER * EMBED_DIM // OUT_MINOR)
        idx_vs = (idx_v0, idx_v1)
        idx_sems = (idx_sem0, idx_sem1)
        gat_sems = (gat_sem0, gat_sem1)
        st_sems = (st_sem0, st_sem1)
        lane = lax.iota(jnp.int32, LANES)
        perms = [(lane & (CLS - 1)) * CLS + k for k in range(CLS)]
        blocks = [(lane >> 2) == m for m in range(CLS)]

        for s in (0, 1):
            pltpu.async_copy(
                idx_hbm.at[pl.ds(base + s * CHUNK, CHUNK)], idx_vs[s],
                idx_sems[s])

        def body(gpair, carry):
            for s in (0, 1):
                g = gpair * 2 + s
                f0 = base + g * CHUNK
                o0 = obase + g * CHUNK_OUT

                pltpu.make_async_copy(
                    idx_hbm.at[pl.ds(f0, CHUNK)], idx_vs[s],
                    idx_sems[s]).wait()

                for j in range(CHUNK // (LANES * CLS)):
                    vs_ = [idx_vs[s][pl.ds((j * CLS + m) * LANES, LANES)]
                           for m in range(CLS)]
                    for k in range(CLS):
                        gk = [vs_[m][perms[k]] for m in range(CLS)]
                        merged = gk[0]
                        for m in range(1, CLS):
                            merged = jnp.where(blocks[m], gk[m], merged)
                        cls_v[s, k, pl.ds(j * LANES, LANES)] = merged

                @pl.when(gpair >= 1)
                def _():
                    for k in range(CLS):
                        pltpu.make_async_copy(
                            rows_v.at[s, k],
                            out_hbm.at[pl.ds(o0 - 2 * CHUNK_OUT, CHUNK_OUT),
                                       pl.ds(EMBED_DIM * k, EMBED_DIM)],
                            st_sems[s]).wait()

                copies = [
                    pltpu.async_copy(
                        table_hbm.at[cls_v.at[s, k]],
                        rows_v.at[s, k],
                        gat_sems[s])
                    for k in range(CLS)
                ]
                for c in copies:
                    c.wait()

                for k in range(CLS):
                    pltpu.async_copy(
                        rows_v.at[s, k],
                        out_hbm.at[pl.ds(o0, CHUNK_OUT),
                                   pl.ds(EMBED_DIM * k, EMBED_DIM)],
                        st_sems[s])

                @pl.when(gpair < NUM_PAIRS - 1)
                def _():
                    pltpu.async_copy(
                        idx_hbm.at[pl.ds(f0 + 2 * CHUNK, CHUNK)],
                        idx_vs[s], idx_sems[s])

            return carry

        lax.fori_loop(0, NUM_PAIRS, body, 0)

        for s in (0, 1):
            g = NUM_CHUNKS - 2 + s
            for k in range(CLS):
                pltpu.make_async_copy(
                    rows_v.at[s, k],
                    out_hbm.at[pl.ds(obase + g * CHUNK_OUT, CHUNK_OUT),
                               pl.ds(EMBED_DIM * k, EMBED_DIM)],
                    st_sems[s]).wait()

    return gather_kernel


_gather = _make_gather()

TC_BB = 2048
TC_NBLK = BATCH // TC_BB
TC_F4 = FIELDS // 4


def _tc_transpose_body(x_ref, o_ref):
    o_ref[...] = x_ref[...].T.reshape(4, EMBED_DIM, TC_BB)


_tc_transpose = pl.pallas_call(
    _tc_transpose_body,
    grid=(TC_F4, TC_NBLK),
    in_specs=[pl.BlockSpec((TC_BB, 128), lambda f, i: (i, f))],
    out_specs=pl.BlockSpec((4, EMBED_DIM, TC_BB), lambda f, i: (f, 0, i)),
    out_shape=jax.ShapeDtypeStruct((FIELDS, EMBED_DIM, BATCH), jnp.float32),
)


def kernel(indices, table):
    out = _gather(indices.reshape(NROWS), table)
    mid = _tc_transpose(out.reshape(BATCH, FIELDS * EMBED_DIM))
    return mid.transpose(2, 0, 1)


# speedup vs baseline: 4.5049x; 1.0615x over previous
import functools

import jax
import jax.numpy as jnp
from jax import lax
from jax.experimental import pallas as pl
from jax.experimental.pallas import tpu as pltpu
from jax.experimental.pallas import tpu_sc as plsc

BATCH = 16384
FIELDS = 100
EMBED_DIM = 32
NROWS = BATCH * FIELDS
NUM_COLUMNS_P1 = 1000001
OUT_MINOR = 128
OUT_MAJOR = NROWS * EMBED_DIM // OUT_MINOR

NUM_CORES = 2
NUM_SUBCORES = 16
NUM_WORKERS = NUM_CORES * NUM_SUBCORES
ROWS_PER_WORKER = NROWS // NUM_WORKERS
CHUNK = 1600
NUM_CHUNKS = ROWS_PER_WORKER // CHUNK
NUM_PAIRS = NUM_CHUNKS // 2
CHUNK_OUT = CHUNK * EMBED_DIM // OUT_MINOR
LANES = 16
CLS = OUT_MINOR // EMBED_DIM
PER_CLS = CHUNK // CLS


def _make_gather():
    mesh = plsc.VectorSubcoreMesh(core_axis_name="c", subcore_axis_name="s")

    @functools.partial(
        pl.kernel,
        mesh=mesh,
        out_type=jax.ShapeDtypeStruct((OUT_MAJOR, OUT_MINOR), jnp.float32),
        scratch_types=[
            pltpu.VMEM((CHUNK,), jnp.int32),
            pltpu.VMEM((CHUNK,), jnp.int32),
            pltpu.VMEM((2, CLS, PER_CLS), jnp.int32),
            pltpu.VMEM((2, CLS, PER_CLS, EMBED_DIM), jnp.float32),
            pltpu.SemaphoreType.DMA,
            pltpu.SemaphoreType.DMA,
            pltpu.SemaphoreType.DMA,
            pltpu.SemaphoreType.DMA,
            pltpu.SemaphoreType.DMA,
            pltpu.SemaphoreType.DMA,
        ],
        compiler_params=pltpu.CompilerParams(use_tc_tiling_on_sc=False),
    )
    def gather_kernel(idx_hbm, table_hbm, out_hbm, idx_v0, idx_v1, cls_v, rows_v,
                      idx_sem0, idx_sem1, gat_sem0, gat_sem1,
                      st_sem0, st_sem1):
        wid = lax.axis_index("s") * NUM_CORES + lax.axis_index("c")
        base = wid * ROWS_PER_WORKER
        obase = wid * (ROWS_PER_WORKER * EMBED_DIM // OUT_MINOR)
        idx_vs = (idx_v0, idx_v1)
        idx_sems = (idx_sem0, idx_sem1)
        gat_sems = (gat_sem0, gat_sem1)
        st_sems = (st_sem0, st_sem1)
        lane = lax.iota(jnp.int32, LANES)
        perms = [(lane & (CLS - 1)) * CLS + k for k in range(CLS)]
        blocks = [(lane >> 2) == m for m in range(CLS)]

        for s in (0, 1):
            pltpu.async_copy(
                idx_hbm.at[pl.ds(base + s * CHUNK, CHUNK)], idx_vs[s],
                idx_sems[s])

        def body(gpair, carry):
            for s in (0, 1):
                g = gpair * 2 + s
                f0 = base + g * CHUNK
                o0 = obase + g * CHUNK_OUT

                pltpu.make_async_copy(
                    idx_hbm.at[pl.ds(f0, CHUNK)], idx_vs[s],
                    idx_sems[s]).wait()

                for j in range(CHUNK // (LANES * CLS)):
                    vs_ = [idx_vs[s][pl.ds((j * CLS + m) * LANES, LANES)]
                           for m in range(CLS)]
                    for k in range(CLS):
                        gk = [vs_[m][perms[k]] for m in range(CLS)]
                        merged = gk[0]
                        for m in range(1, CLS):
                            merged = jnp.where(blocks[m], gk[m], merged)
                        cls_v[s, k, pl.ds(j * LANES, LANES)] = merged

                @pl.when(gpair >= 1)
                def _():
                    for k in range(CLS):
                        pltpu.make_async_copy(
                            rows_v.at[s, k],
                            out_hbm.at[pl.ds(o0 - 2 * CHUNK_OUT, CHUNK_OUT),
                                       pl.ds(EMBED_DIM * k, EMBED_DIM)],
                            st_sems[s]).wait()

                copies = [
                    pltpu.async_copy(
                        table_hbm.at[cls_v.at[s, k]],
                        rows_v.at[s, k],
                        gat_sems[s])
                    for k in range(CLS)
                ]
                for c in copies:
                    c.wait()

                for k in range(CLS):
                    pltpu.async_copy(
                        rows_v.at[s, k],
                        out_hbm.at[pl.ds(o0, CHUNK_OUT),
                                   pl.ds(EMBED_DIM * k, EMBED_DIM)],
                        st_sems[s])

                @pl.when(gpair < NUM_PAIRS - 1)
                def _():
                    pltpu.async_copy(
                        idx_hbm.at[pl.ds(f0 + 2 * CHUNK, CHUNK)],
                        idx_vs[s], idx_sems[s])

            return carry

        lax.fori_loop(0, NUM_PAIRS, body, 0)

        for s in (0, 1):
            g = NUM_CHUNKS - 2 + s
            for k in range(CLS):
                pltpu.make_async_copy(
                    rows_v.at[s, k],
                    out_hbm.at[pl.ds(obase + g * CHUNK_OUT, CHUNK_OUT),
                               pl.ds(EMBED_DIM * k, EMBED_DIM)],
                    st_sems[s]).wait()

    return gather_kernel


_gather = _make_gather()

TC_BB = 4096
TC_NBLK = BATCH // TC_BB
TC_F4 = FIELDS // 4


def _tc_transpose_body(x_ref, o_ref):
    o_ref[...] = x_ref[...].T.reshape(4, EMBED_DIM, TC_BB)


_tc_transpose = pl.pallas_call(
    _tc_transpose_body,
    grid=(TC_F4, TC_NBLK),
    in_specs=[pl.BlockSpec((TC_BB, 128), lambda f, i: (i, f))],
    out_specs=pl.BlockSpec((4, EMBED_DIM, TC_BB), lambda f, i: (f, 0, i)),
    out_shape=jax.ShapeDtypeStruct((FIELDS, EMBED_DIM, BATCH), jnp.float32),
)


def kernel(indices, table):
    out = _gather(indices.reshape(NROWS), table)
    mid = _tc_transpose(out.reshape(BATCH, FIELDS * EMBED_DIM))
    return mid.transpose(2, 0, 1)


# speedup vs baseline: 4.6047x; 1.0222x over previous
import functools

import jax
import jax.numpy as jnp
from jax import lax
from jax.experimental import pallas as pl
from jax.experimental.pallas import tpu as pltpu
from jax.experimental.pallas import tpu_sc as plsc

BATCH = 16384
FIELDS = 100
EMBED_DIM = 32
NROWS = BATCH * FIELDS
NUM_COLUMNS_P1 = 1000001
OUT_MINOR = 128
OUT_MAJOR = NROWS * EMBED_DIM // OUT_MINOR

NUM_CORES = 2
NUM_SUBCORES = 16
NUM_WORKERS = NUM_CORES * NUM_SUBCORES
ROWS_PER_WORKER = NROWS // NUM_WORKERS
CHUNK = 1600
NUM_CHUNKS = ROWS_PER_WORKER // CHUNK
NUM_PAIRS = NUM_CHUNKS // 2
CHUNK_OUT = CHUNK * EMBED_DIM // OUT_MINOR
LANES = 16
CLS = OUT_MINOR // EMBED_DIM
PER_CLS = CHUNK // CLS


def _make_gather():
    mesh = plsc.VectorSubcoreMesh(core_axis_name="c", subcore_axis_name="s")

    @functools.partial(
        pl.kernel,
        mesh=mesh,
        out_type=jax.ShapeDtypeStruct((OUT_MAJOR, OUT_MINOR), jnp.float32),
        scratch_types=[
            pltpu.VMEM((CHUNK,), jnp.int32),
            pltpu.VMEM((CHUNK,), jnp.int32),
            pltpu.VMEM((2, CLS, PER_CLS), jnp.int32),
            pltpu.VMEM((2, CLS, PER_CLS, EMBED_DIM), jnp.float32),
            pltpu.SemaphoreType.DMA,
            pltpu.SemaphoreType.DMA,
            pltpu.SemaphoreType.DMA,
            pltpu.SemaphoreType.DMA,
            pltpu.SemaphoreType.DMA,
            pltpu.SemaphoreType.DMA,
        ],
        compiler_params=pltpu.CompilerParams(use_tc_tiling_on_sc=False),
    )
    def gather_kernel(idx_hbm, table_hbm, out_hbm, idx_v0, idx_v1, cls_v, rows_v,
                      idx_sem0, idx_sem1, gat_sem0, gat_sem1,
                      st_sem0, st_sem1):
        wid = lax.axis_index("s") * NUM_CORES + lax.axis_index("c")
        base = wid * ROWS_PER_WORKER
        obase = wid * (ROWS_PER_WORKER * EMBED_DIM // OUT_MINOR)
        idx_vs = (idx_v0, idx_v1)
        idx_sems = (idx_sem0, idx_sem1)
        gat_sems = (gat_sem0, gat_sem1)
        st_sems = (st_sem0, st_sem1)
        lane = lax.iota(jnp.int32, LANES)
        perms = [(lane & (CLS - 1)) * CLS + k for k in range(CLS)]
        blocks = [(lane >> 2) == m for m in range(CLS)]

        for s in (0, 1):
            pltpu.async_copy(
                idx_hbm.at[pl.ds(base + s * CHUNK, CHUNK)], idx_vs[s],
                idx_sems[s])

        def body(gpair, carry):
            for s in (0, 1):
                g = gpair * 2 + s
                f0 = base + g * CHUNK
                o0 = obase + g * CHUNK_OUT

                pltpu.make_async_copy(
                    idx_hbm.at[pl.ds(f0, CHUNK)], idx_vs[s],
                    idx_sems[s]).wait()

                for j in range(CHUNK // (LANES * CLS)):
                    vs_ = [idx_vs[s][pl.ds((j * CLS + m) * LANES, LANES)]
                           for m in range(CLS)]
                    for k in range(CLS):
                        gk = [vs_[m][perms[k]] for m in range(CLS)]
                        merged = gk[0]
                        for m in range(1, CLS):
                            merged = jnp.where(blocks[m], gk[m], merged)
                        cls_v[s, k, pl.ds(j * LANES, LANES)] = merged

                @pl.when(gpair >= 1)
                def _():
                    for k in range(CLS):
                        pltpu.make_async_copy(
                            rows_v.at[s, k],
                            out_hbm.at[pl.ds(o0 - 2 * CHUNK_OUT, CHUNK_OUT),
                                       pl.ds(EMBED_DIM * k, EMBED_DIM)],
                            st_sems[s]).wait()

                copies = [
                    pltpu.async_copy(
                        table_hbm.at[cls_v.at[s, k]],
                        rows_v.at[s, k],
                        gat_sems[s])
                    for k in range(CLS)
                ]
                for c in copies:
                    c.wait()

                for k in range(CLS):
                    pltpu.async_copy(
                        rows_v.at[s, k],
                        out_hbm.at[pl.ds(o0, CHUNK_OUT),
                                   pl.ds(EMBED_DIM * k, EMBED_DIM)],
                        st_sems[s])

                @pl.when(gpair < NUM_PAIRS - 1)
                def _():
                    pltpu.async_copy(
                        idx_hbm.at[pl.ds(f0 + 2 * CHUNK, CHUNK)],
                        idx_vs[s], idx_sems[s])

            return carry

        lax.fori_loop(0, NUM_PAIRS, body, 0)

        for s in (0, 1):
            g = NUM_CHUNKS - 2 + s
            for k in range(CLS):
                pltpu.make_async_copy(
                    rows_v.at[s, k],
                    out_hbm.at[pl.ds(obase + g * CHUNK_OUT, CHUNK_OUT),
                               pl.ds(EMBED_DIM * k, EMBED_DIM)],
                    st_sems[s]).wait()

    return gather_kernel


_gather = _make_gather()

TC_BB = 8192
TC_NBLK = BATCH // TC_BB
TC_F4 = FIELDS // 4


def _tc_transpose_body(x_ref, o_ref):
    o_ref[...] = x_ref[...].T.reshape(4, EMBED_DIM, TC_BB)


_tc_transpose = pl.pallas_call(
    _tc_transpose_body,
    grid=(TC_F4, TC_NBLK),
    in_specs=[pl.BlockSpec((TC_BB, 128), lambda f, i: (i, f))],
    out_specs=pl.BlockSpec((4, EMBED_DIM, TC_BB), lambda f, i: (f, 0, i)),
    out_shape=jax.ShapeDtypeStruct((FIELDS, EMBED_DIM, BATCH), jnp.float32),
)


def kernel(indices, table):
    out = _gather(indices.reshape(NROWS), table)
    mid = _tc_transpose(out.reshape(BATCH, FIELDS * EMBED_DIM))
    return mid.transpose(2, 0, 1)


# speedup vs baseline: 4.6283x; 1.0051x over previous
import functools

import jax
import jax.numpy as jnp
from jax import lax
from jax.experimental import pallas as pl
from jax.experimental.pallas import tpu as pltpu
from jax.experimental.pallas import tpu_sc as plsc

BATCH = 16384
FIELDS = 100
EMBED_DIM = 32
NROWS = BATCH * FIELDS
NUM_COLUMNS_P1 = 1000001
OUT_MINOR = 128
OUT_MAJOR = NROWS * EMBED_DIM // OUT_MINOR

NUM_CORES = 2
NUM_SUBCORES = 16
NUM_WORKERS = NUM_CORES * NUM_SUBCORES
ROWS_PER_WORKER = NROWS // NUM_WORKERS
CHUNK = 1600
NUM_CHUNKS = ROWS_PER_WORKER // CHUNK
NUM_PAIRS = NUM_CHUNKS // 2
CHUNK_OUT = CHUNK * EMBED_DIM // OUT_MINOR
LANES = 16
CLS = OUT_MINOR // EMBED_DIM
PER_CLS = CHUNK // CLS


def _make_gather():
    mesh = plsc.VectorSubcoreMesh(core_axis_name="c", subcore_axis_name="s")

    @functools.partial(
        pl.kernel,
        mesh=mesh,
        out_type=jax.ShapeDtypeStruct((OUT_MAJOR, OUT_MINOR), jnp.float32),
        scratch_types=[
            pltpu.VMEM((CHUNK,), jnp.int32),
            pltpu.VMEM((CHUNK,), jnp.int32),
            pltpu.VMEM((2, CLS, PER_CLS), jnp.int32),
            pltpu.VMEM((2, CLS, PER_CLS, EMBED_DIM), jnp.float32),
            pltpu.SemaphoreType.DMA,
            pltpu.SemaphoreType.DMA,
            pltpu.SemaphoreType.DMA,
            pltpu.SemaphoreType.DMA,
            pltpu.SemaphoreType.DMA,
            pltpu.SemaphoreType.DMA,
        ],
        compiler_params=pltpu.CompilerParams(use_tc_tiling_on_sc=False),
    )
    def gather_kernel(idx_hbm, table_hbm, out_hbm, idx_v0, idx_v1, cls_v, rows_v,
                      idx_sem0, idx_sem1, gat_sem0, gat_sem1,
                      st_sem0, st_sem1):
        wid = lax.axis_index("s") * NUM_CORES + lax.axis_index("c")
        base = wid * ROWS_PER_WORKER
        obase = wid * (ROWS_PER_WORKER * EMBED_DIM // OUT_MINOR)
        idx_vs = (idx_v0, idx_v1)
        idx_sems = (idx_sem0, idx_sem1)
        gat_sems = (gat_sem0, gat_sem1)
        st_sems = (st_sem0, st_sem1)
        lane = lax.iota(jnp.int32, LANES)
        perms = [(lane & (CLS - 1)) * CLS + k for k in range(CLS)]
        blocks = [(lane >> 2) == m for m in range(CLS)]

        for s in (0, 1):
            pltpu.async_copy(
                idx_hbm.at[pl.ds(base + s * CHUNK, CHUNK)], idx_vs[s],
                idx_sems[s])

        def body(gpair, carry):
            for s in (0, 1):
                g = gpair * 2 + s
                f0 = base + g * CHUNK
                o0 = obase + g * CHUNK_OUT

                pltpu.make_async_copy(
                    idx_hbm.at[pl.ds(f0, CHUNK)], idx_vs[s],
                    idx_sems[s]).wait()

                for j in range(CHUNK // (LANES * CLS)):
                    vs_ = [idx_vs[s][pl.ds((j * CLS + m) * LANES, LANES)]
                           for m in range(CLS)]
                    for k in range(CLS):
                        gk = [vs_[m][perms[k]] for m in range(CLS)]
                        merged = gk[0]
                        for m in range(1, CLS):
                            merged = jnp.where(blocks[m], gk[m], merged)
                        cls_v[s, k, pl.ds(j * LANES, LANES)] = merged

                @pl.when(gpair >= 1)
                def _():
                    for k in range(CLS):
                        pltpu.make_async_copy(
                            rows_v.at[s, k],
                            out_hbm.at[pl.ds(o0 - 2 * CHUNK_OUT, CHUNK_OUT),
                                       pl.ds(EMBED_DIM * k, EMBED_DIM)],
                            st_sems[s]).wait()

                copies = [
                    pltpu.async_copy(
                        table_hbm.at[cls_v.at[s, k]],
                        rows_v.at[s, k],
                        gat_sems[s])
                    for k in range(CLS)
                ]
                for c in copies:
                    c.wait()

                for k in range(CLS):
                    pltpu.async_copy(
                        rows_v.at[s, k],
                        out_hbm.at[pl.ds(o0, CHUNK_OUT),
                                   pl.ds(EMBED_DIM * k, EMBED_DIM)],
                        st_sems[s])

                @pl.when(gpair < NUM_PAIRS - 1)
                def _():
                    pltpu.async_copy(
                        idx_hbm.at[pl.ds(f0 + 2 * CHUNK, CHUNK)],
                        idx_vs[s], idx_sems[s])

            return carry

        lax.fori_loop(0, NUM_PAIRS, body, 0)

        for s in (0, 1):
            g = NUM_CHUNKS - 2 + s
            for k in range(CLS):
                pltpu.make_async_copy(
                    rows_v.at[s, k],
                    out_hbm.at[pl.ds(obase + g * CHUNK_OUT, CHUNK_OUT),
                               pl.ds(EMBED_DIM * k, EMBED_DIM)],
                    st_sems[s]).wait()

    return gather_kernel


_gather = _make_gather()

TC_BB = 16384
TC_NBLK = BATCH // TC_BB
TC_F4 = FIELDS // 4


def _tc_transpose_body(x_ref, o_ref):
    o_ref[...] = x_ref[...].T.reshape(4, EMBED_DIM, TC_BB)


_tc_transpose = pl.pallas_call(
    _tc_transpose_body,
    grid=(TC_F4, TC_NBLK),
    in_specs=[pl.BlockSpec((TC_BB, 128), lambda f, i: (i, f))],
    out_specs=pl.BlockSpec((4, EMBED_DIM, TC_BB), lambda f, i: (f, 0, i)),
    out_shape=jax.ShapeDtypeStruct((FIELDS, EMBED_DIM, BATCH), jnp.float32),
)


def kernel(indices, table):
    out = _gather(indices.reshape(NROWS), table)
    mid = _tc_transpose(out.reshape(BATCH, FIELDS * EMBED_DIM))
    return mid.transpose(2, 0, 1)


# speedup vs baseline: 4.6323x; 1.0009x over previous
import functools

import jax
import jax.numpy as jnp
from jax import lax
from jax.experimental import pallas as pl
from jax.experimental.pallas import tpu as pltpu
from jax.experimental.pallas import tpu_sc as plsc

BATCH = 16384
FIELDS = 100
EMBED_DIM = 32
NROWS = BATCH * FIELDS
OUT_MINOR = 128
OUT_MAJOR = NROWS * EMBED_DIM // OUT_MINOR

NUM_CORES = 2
NUM_SUBCORES = 16
NUM_WORKERS = NUM_CORES * NUM_SUBCORES
ROWS_PER_WORKER = NROWS // NUM_WORKERS
CHUNK = 1600
NUM_CHUNKS = ROWS_PER_WORKER // CHUNK
NUM_PAIRS = NUM_CHUNKS // 2
CHUNK_OUT = CHUNK * EMBED_DIM // OUT_MINOR
LANES = 16
CLS = OUT_MINOR // EMBED_DIM
PER_CLS = CHUNK // CLS


def _make_gather():
    mesh = plsc.VectorSubcoreMesh(core_axis_name="c", subcore_axis_name="s")

    @functools.partial(
        pl.kernel,
        mesh=mesh,
        out_type=jax.ShapeDtypeStruct((OUT_MAJOR, OUT_MINOR), jnp.float32),
        scratch_types=[
            pltpu.VMEM((CHUNK,), jnp.int32),
            pltpu.VMEM((CHUNK,), jnp.int32),
            pltpu.VMEM((2, CLS, PER_CLS), jnp.int32),
            pltpu.VMEM((2, CLS, PER_CLS, EMBED_DIM), jnp.float32),
            pltpu.SemaphoreType.DMA,
            pltpu.SemaphoreType.DMA,
            pltpu.SemaphoreType.DMA,
            pltpu.SemaphoreType.DMA,
            pltpu.SemaphoreType.DMA,
            pltpu.SemaphoreType.DMA,
        ],
        compiler_params=pltpu.CompilerParams(use_tc_tiling_on_sc=False),
    )
    def gather_kernel(idx_hbm, table_hbm, out_hbm, idx_v0, idx_v1, cls_v, rows_v,
                      idx_sem0, idx_sem1, gat_sem0, gat_sem1,
                      st_sem0, st_sem1):
        wid = lax.axis_index("s") * NUM_CORES + lax.axis_index("c")
        base = wid * ROWS_PER_WORKER
        obase = wid * (ROWS_PER_WORKER * EMBED_DIM // OUT_MINOR)
        idx_vs = (idx_v0, idx_v1)
        idx_sems = (idx_sem0, idx_sem1)
        gat_sems = (gat_sem0, gat_sem1)
        st_sems = (st_sem0, st_sem1)
        lane = lax.iota(jnp.int32, LANES)
        perms = [(lane & (CLS - 1)) * CLS + k for k in range(CLS)]
        blocks = [(lane >> 2) == m for m in range(CLS)]

        for s in (0, 1):
            pltpu.async_copy(
                idx_hbm.at[pl.ds(base + s * CHUNK, CHUNK)], idx_vs[s],
                idx_sems[s])

        def body(gpair, carry):
            for s in (0, 1):
                g = gpair * 2 + s
                f0 = base + g * CHUNK
                o0 = obase + g * CHUNK_OUT

                pltpu.make_async_copy(
                    idx_hbm.at[pl.ds(f0, CHUNK)], idx_vs[s],
                    idx_sems[s]).wait()

                for j in range(CHUNK // (LANES * CLS)):
                    vs_ = [idx_vs[s][pl.ds((j * CLS + m) * LANES, LANES)]
                           for m in range(CLS)]
                    for k in range(CLS):
                        gk = [vs_[m][perms[k]] for m in range(CLS)]
                        merged = gk[0]
                        for m in range(1, CLS):
                            merged = jnp.where(blocks[m], gk[m], merged)
                        cls_v[s, k, pl.ds(j * LANES, LANES)] = merged

                @pl.when(gpair >= 1)
                def _():
                    for k in range(CLS):
                        pltpu.make_async_copy(
                            rows_v.at[s, k],
                            out_hbm.at[pl.ds(o0 - 2 * CHUNK_OUT, CHUNK_OUT),
                                       pl.ds(EMBED_DIM * k, EMBED_DIM)],
                            st_sems[s]).wait()

                copies = [
                    pltpu.async_copy(
                        table_hbm.at[cls_v.at[s, k]],
                        rows_v.at[s, k],
                        gat_sems[s])
                    for k in range(CLS)
                ]
                for c in copies:
                    c.wait()

                for k in range(CLS):
                    pltpu.async_copy(
                        rows_v.at[s, k],
                        out_hbm.at[pl.ds(o0, CHUNK_OUT),
                                   pl.ds(EMBED_DIM * k, EMBED_DIM)],
                        st_sems[s])

                @pl.when(gpair < NUM_PAIRS - 1)
                def _():
                    pltpu.async_copy(
                        idx_hbm.at[pl.ds(f0 + 2 * CHUNK, CHUNK)],
                        idx_vs[s], idx_sems[s])

            return carry

        lax.fori_loop(0, NUM_PAIRS, body, 0)

        for s in (0, 1):
            g = NUM_CHUNKS - 2 + s
            for k in range(CLS):
                pltpu.make_async_copy(
                    rows_v.at[s, k],
                    out_hbm.at[pl.ds(obase + g * CHUNK_OUT, CHUNK_OUT),
                               pl.ds(EMBED_DIM * k, EMBED_DIM)],
                    st_sems[s]).wait()

    return gather_kernel


_gather = _make_gather()

TC_BB = 16384
TC_NBLK = BATCH // TC_BB
TC_F4 = FIELDS // 4


def _tc_transpose_body(x_ref, o_ref):
    o_ref[...] = x_ref[...].T.reshape(4, EMBED_DIM, TC_BB)


_tc_transpose = pl.pallas_call(
    _tc_transpose_body,
    grid=(TC_F4, TC_NBLK),
    in_specs=[pl.BlockSpec((TC_BB, 128), lambda f, i: (i, f))],
    out_specs=pl.BlockSpec((4, EMBED_DIM, TC_BB), lambda f, i: (f, 0, i)),
    out_shape=jax.ShapeDtypeStruct((FIELDS, EMBED_DIM, BATCH), jnp.float32),
)


def kernel(indices, table):
    out = _gather(indices.reshape(NROWS), table)
    mid = _tc_transpose(out.reshape(BATCH, FIELDS * EMBED_DIM))
    return mid.transpose(2, 0, 1)
